# baseline (speedup 1.0000x reference)
import functools

import jax
import jax.numpy as jnp
from jax import lax
from jax.experimental import pallas as pl
from jax.experimental.pallas import tpu as pltpu
from jax.experimental.pallas import tpu_sc as plsc

BATCH = 32
BEAM = 8
VOCAB = 100000
EOS_INDEX = 2
EOS_THRESHOLD = 1.5
MINUS_INF = -1e20
NEG = -3.0e38

CHUNK = 6400
NCHUNK = 15
ECHUNK = 3968
ALIGNED = NCHUNK * CHUNK + ECHUNK
TAIL = VOCAB - ALIGNED


def _lane_iota():
    return lax.iota(jnp.int32, 16)


def _tree_max(vs):
    while len(vs) > 1:
        nxt = [jnp.maximum(vs[i], vs[i + 1]) for i in range(0, len(vs) - 1, 2)]
        if len(vs) % 2:
            nxt.append(vs[-1])
        vs = nxt
    return vs[0]


def _merge(T_val, T_idx, n_val, n_idx):
    s_val, s_idx = plsc.sort_key_val(n_val, n_idx)
    take_t = T_val >= s_val
    u_val = jnp.where(take_t, T_val, s_val)
    u_idx = jnp.where(take_t, T_idx, s_idx)
    return plsc.sort_key_val(u_val, u_idx, descending=True)


def _theta_of(T_val):
    return jnp.max(jnp.where(_lane_iota() == 7, T_val, NEG))


def _body(lp_hbm, seq_hbm, out_s, out_t, out_p,
          buf, tailbuf, seqbuf, ov, ot, op, sems, osem):
    wid = lax.axis_index("s") * 2 + lax.axis_index("c")
    row0 = wid * BEAM

    pltpu.sync_copy(seq_hbm, seqbuf.at[pl.ds(0, BATCH * BEAM)])
    pltpu.sync_copy(lp_hbm.at[pl.ds(row0, BEAM), pl.ds(ALIGNED, TAIL)],
                    tailbuf)
    pltpu.async_copy(lp_hbm.at[pl.ds(row0, BEAM), pl.ds(0, CHUNK)],
                     buf.at[0], sems.at[0])

    lanes = _lane_iota()
    sv = seqbuf[pl.ds(row0, 16)]

    def s_of(r):
        return jnp.max(jnp.where(lanes == r, sv, NEG))

    def scan_sub(sb, carry, parity, r, s_r, idx_base0, sub, grp):
        T_val, T_idx, theta, rowacc = carry
        base = sb * (sub * 16)
        vs = [buf[parity, r, pl.ds(base + k * 16, 16)] for k in range(sub)]
        nacc = min(8, sub)
        accs = list(vs[:nacc])
        for k in range(nacc, sub):
            accs[k % nacc] = jnp.maximum(accs[k % nacc], vs[k])
        m = jnp.max(_tree_max(accs))
        rowacc = jnp.maximum(rowacc, m)

        def do_merge(T_val, T_idx, theta):
            for g0 in range(0, sub, grp):
                gn = min(grp, sub - g0)
                gb = base + g0 * 16
                gvs = [buf[parity, r, pl.ds(gb + k * 16, 16)]
                       for k in range(gn)]
                gm = jnp.max(_tree_max(gvs))

                def merge_grp(T_val, T_idx, theta, gvs=gvs, gb=gb, gn=gn):
                    tv, ti = T_val, T_idx
                    sb16 = jnp.full((16,), s_r)
                    for k in range(gn):
                        val = gvs[k] + sb16
                        idx = (idx_base0 + gb + k * 16) + lanes
                        tv, ti = _merge(tv, ti, val, idx)
                    return tv, ti, _theta_of(tv)

                T_val, T_idx, theta = lax.cond(
                    gm + s_r > theta, merge_grp,
                    lambda tv, ti, th: (tv, ti, th),
                    T_val, T_idx, theta)
            return T_val, T_idx, theta

        T_val, T_idx, theta = lax.cond(
            m + s_r > theta, do_merge,
            lambda tv, ti, th: (tv, ti, th),
            T_val, T_idx, theta)
        return T_val, T_idx, theta, rowacc

    def scan_row(r, carry, parity, col0, ncols, sub, grp):
        T_val, T_idx, theta, rowmax = carry
        s_r = s_of(r)
        idx_base0 = r * VOCAB + col0
        T_val, T_idx, theta, rowacc = lax.fori_loop(
            0, ncols // (sub * 16),
            functools.partial(scan_sub, parity=parity, r=r, s_r=s_r,
                              idx_base0=idx_base0, sub=sub, grp=grp),
            (T_val, T_idx, theta, jnp.float32(NEG)))
        rowmax = jnp.where(lanes == r, jnp.maximum(rowmax, rowacc), rowmax)
        return T_val, T_idx, theta, rowmax

    def step(t, carry):
        T_val, T_idx, theta, rowmax = carry
        parity = t % 2

        @pl.when(t + 1 < NCHUNK)
        def _start_next_uniform():
            pltpu.async_copy(
                lp_hbm.at[pl.ds(row0, BEAM), pl.ds((t + 1) * CHUNK, CHUNK)],
                buf.at[(t + 1) % 2], sems.at[(t + 1) % 2])

        @pl.when(t + 1 == NCHUNK)
        def _start_epilogue():
            pltpu.async_copy(
                lp_hbm.at[pl.ds(row0, BEAM), pl.ds(NCHUNK * CHUNK, ECHUNK)],
                buf.at[(t + 1) % 2, :, pl.ds(0, ECHUNK)],
                sems.at[(t + 1) % 2])

        @pl.when(t > 0)
        def _wait_chunk():
            pltpu.make_async_copy(
                lp_hbm.at[pl.ds(row0, BEAM), pl.ds(t * CHUNK, CHUNK)],
                buf.at[parity], sems.at[parity]).wait()

        T_val, T_idx, theta, rowmax = lax.fori_loop(
            0, BEAM,
            functools.partial(scan_row, parity=parity, col0=t * CHUNK,
                              ncols=CHUNK, sub=200, grp=10),
            (T_val, T_idx, theta, rowmax))
        return T_val, T_idx, theta, rowmax

    init = (jnp.full((16,), NEG, jnp.float32),
            jnp.zeros((16,), jnp.int32),
            jnp.float32(NEG),
            jnp.full((16,), NEG, jnp.float32))

    pltpu.make_async_copy(
        lp_hbm.at[pl.ds(row0, BEAM), pl.ds(0, CHUNK)],
        buf.at[0], sems.at[0]).wait()
    eos_vec = jnp.full((16,), NEG, jnp.float32)
    for r in range(BEAM):
        v0 = buf[0, r, pl.ds(0, 16)]
        e_r = jnp.max(jnp.where(lanes == EOS_INDEX, v0, NEG))
        eos_vec = jnp.where(lanes == r, jnp.full((16,), e_r), eos_vec)
        buf[0, r, pl.ds(0, 16)] = jnp.where(
            lanes == EOS_INDEX, jnp.full((16,), NEG, jnp.float32), v0)

    T_val, T_idx, theta, rowmax = lax.fori_loop(0, NCHUNK, step, init)

    ep = NCHUNK % 2
    pltpu.make_async_copy(
        lp_hbm.at[pl.ds(row0, BEAM), pl.ds(NCHUNK * CHUNK, ECHUNK)],
        buf.at[ep, :, pl.ds(0, ECHUNK)], sems.at[ep]).wait()
    T_val, T_idx, theta, rowmax = lax.fori_loop(
        0, BEAM,
        functools.partial(scan_row, parity=ep, col0=NCHUNK * CHUNK,
                          ncols=ECHUNK, sub=31, grp=31),
        (T_val, T_idx, theta, rowmax))

    def tail_row(r, carry):
        T_val, T_idx, theta, rowmax = carry
        s_r = s_of(r)
        v0 = tailbuf[r, pl.ds(0, 16)]
        v1 = tailbuf[r, pl.ds(16, 16)]
        m = jnp.max(jnp.maximum(v0, v1))
        rowmax = jnp.where(lanes == r, jnp.maximum(rowmax, m), rowmax)

        def do_merge(T_val, T_idx, theta):
            tv, ti = T_val, T_idx
            sb16 = jnp.full((16,), s_r)
            base = r * VOCAB + ALIGNED
            tv, ti = _merge(tv, ti, v0 + sb16, base + lanes)
            tv, ti = _merge(tv, ti, v1 + sb16, base + 16 + lanes)
            return tv, ti, _theta_of(tv)

        T_val, T_idx, theta = lax.cond(
            m + s_r > theta, do_merge,
            lambda tv, ti, th: (tv, ti, th), T_val, T_idx, theta)
        return T_val, T_idx, theta, rowmax

    T_val, T_idx, theta, rowmax = lax.fori_loop(
        0, BEAM, tail_row, (T_val, T_idx, theta, rowmax))

    true_rowmax = jnp.maximum(rowmax, eos_vec)
    keep = eos_vec > EOS_THRESHOLD * true_rowmax
    eos_vals = jnp.where(
        jnp.logical_and(lanes < BEAM, keep), eos_vec,
        jnp.where(lanes < BEAM, jnp.full((16,), jnp.float32(MINUS_INF)),
                  jnp.full((16,), NEG, jnp.float32)))
    eos_vals = jnp.where(lanes < BEAM, eos_vals + sv, eos_vals)
    eos_idx = lanes * VOCAB + EOS_INDEX
    T_val, T_idx = _merge(T_val, T_idx, eos_vals, eos_idx)

    ov[...] = T_val
    ot[...] = T_idx % VOCAB
    op[...] = T_idx // VOCAB
    pltpu.sync_copy(ov, out_s.at[pl.ds(wid * 16, 16)])
    pltpu.sync_copy(ot, out_t.at[pl.ds(wid * 16, 16)])
    pltpu.sync_copy(op, out_p.at[pl.ds(wid * 16, 16)])


@jax.jit
def kernel(log_probs, sequence_scores):
    mesh = plsc.VectorSubcoreMesh(core_axis_name="c", subcore_axis_name="s")
    f = pl.kernel(
        _body,
        mesh=mesh,
        compiler_params=pltpu.CompilerParams(needs_layout_passes=False),
        out_type=(
            jax.ShapeDtypeStruct((BATCH * 16,), jnp.float32),
            jax.ShapeDtypeStruct((BATCH * 16,), jnp.int32),
            jax.ShapeDtypeStruct((BATCH * 16,), jnp.int32),
        ),
        scratch_types=[
            pltpu.VMEM((2, BEAM, CHUNK), jnp.float32),
            pltpu.VMEM((BEAM, TAIL), jnp.float32),
            pltpu.VMEM((BATCH * BEAM + 16,), jnp.float32),
            pltpu.VMEM((16,), jnp.float32),
            pltpu.VMEM((16,), jnp.int32),
            pltpu.VMEM((16,), jnp.int32),
            pltpu.SemaphoreType.DMA((2,)),
            pltpu.SemaphoreType.DMA,
        ],
    )
    scores, toks, preds = f(log_probs, sequence_scores)
    scores = scores.reshape(BATCH, 16)[:, :BEAM]
    toks = toks.reshape(BATCH, 16)[:, :BEAM]
    preds = preds.reshape(BATCH, 16)[:, :BEAM]
    return scores, toks, preds


# speedup vs baseline: 1.4057x; 1.4057x over previous
import functools

import jax
import jax.numpy as jnp
from jax import lax
from jax.experimental import pallas as pl
from jax.experimental.pallas import tpu as pltpu
from jax.experimental.pallas import tpu_sc as plsc

BATCH = 32
BEAM = 8
VOCAB = 100000
EOS_INDEX = 2
EOS_THRESHOLD = 1.5
MINUS_INF = -1e20
NEG = -3.0e38

CHUNK = 6400
NCHUNK = 15
ECHUNK = 3968
ALIGNED = NCHUNK * CHUNK + ECHUNK
TAIL = VOCAB - ALIGNED


def _lane_iota():
    return lax.iota(jnp.int32, 16)


def _tree_max(vs):
    while len(vs) > 1:
        nxt = [jnp.maximum(vs[i], vs[i + 1]) for i in range(0, len(vs) - 1, 2)]
        if len(vs) % 2:
            nxt.append(vs[-1])
        vs = nxt
    return vs[0]


def _merge(T_val, T_idx, n_val, n_idx):
    s_val, s_idx = plsc.sort_key_val(n_val, n_idx)
    take_t = T_val >= s_val
    u_val = jnp.where(take_t, T_val, s_val)
    u_idx = jnp.where(take_t, T_idx, s_idx)
    return plsc.sort_key_val(u_val, u_idx, descending=True)


def _theta_of(T_val):
    return jnp.max(jnp.where(_lane_iota() == 7, T_val, NEG))


def _body(lp_hbm, seq_hbm, out_s, out_t, out_p,
          buf, tailbuf, seqbuf, ov, ot, op, sems, osem):
    wid = lax.axis_index("s") * 2 + lax.axis_index("c")
    row0 = wid * BEAM

    pltpu.sync_copy(seq_hbm, seqbuf.at[pl.ds(0, BATCH * BEAM)])
    pltpu.sync_copy(lp_hbm.at[pl.ds(row0, BEAM), pl.ds(ALIGNED, TAIL)],
                    tailbuf)
    pltpu.async_copy(lp_hbm.at[pl.ds(row0, BEAM), pl.ds(0, CHUNK)],
                     buf.at[0], sems.at[0])

    lanes = _lane_iota()
    sv = seqbuf[pl.ds(row0, 16)]

    def s_of(r):
        return jnp.max(jnp.where(lanes == r, sv, NEG))

    def scan_sub(sb, carry, parity, r, s_r, idx_base0, sub, grp):
        T_val, T_idx, theta, rowacc = carry
        base = sb * (sub * 16)
        vs = [buf[parity, r, pl.ds(base + k * 16, 16)] for k in range(sub)]
        m = jnp.max(_tree_max(vs))
        rowacc = jnp.maximum(rowacc, m)

        def do_merge(T_val, T_idx, theta):
            for g0 in range(0, sub, grp):
                gn = min(grp, sub - g0)
                gb = base + g0 * 16
                gvs = [buf[parity, r, pl.ds(gb + k * 16, 16)]
                       for k in range(gn)]
                gm = jnp.max(_tree_max(gvs))

                def merge_grp(T_val, T_idx, theta, gvs=gvs, gb=gb, gn=gn):
                    tv, ti = T_val, T_idx
                    sb16 = jnp.full((16,), s_r)
                    for k in range(gn):
                        val = gvs[k] + sb16
                        idx = (idx_base0 + gb + k * 16) + lanes
                        tv, ti = _merge(tv, ti, val, idx)
                    return tv, ti, _theta_of(tv)

                T_val, T_idx, theta = lax.cond(
                    gm + s_r > theta, merge_grp,
                    lambda tv, ti, th: (tv, ti, th),
                    T_val, T_idx, theta)
            return T_val, T_idx, theta

        T_val, T_idx, theta = lax.cond(
            m + s_r > theta, do_merge,
            lambda tv, ti, th: (tv, ti, th),
            T_val, T_idx, theta)
        return T_val, T_idx, theta, rowacc

    def scan_row(r, carry, parity, col0, ncols, sub, grp):
        T_val, T_idx, theta, rowmax = carry
        s_r = s_of(r)
        idx_base0 = r * VOCAB + col0
        T_val, T_idx, theta, rowacc = lax.fori_loop(
            0, ncols // (sub * 16),
            functools.partial(scan_sub, parity=parity, r=r, s_r=s_r,
                              idx_base0=idx_base0, sub=sub, grp=grp),
            (T_val, T_idx, theta, jnp.float32(NEG)))
        rowmax = jnp.where(lanes == r, jnp.maximum(rowmax, rowacc), rowmax)
        return T_val, T_idx, theta, rowmax

    def step(t, carry):
        T_val, T_idx, theta, rowmax = carry
        parity = t % 2

        @pl.when(t + 1 < NCHUNK)
        def _start_next_uniform():
            pltpu.async_copy(
                lp_hbm.at[pl.ds(row0, BEAM), pl.ds((t + 1) * CHUNK, CHUNK)],
                buf.at[(t + 1) % 2], sems.at[(t + 1) % 2])

        @pl.when(t + 1 == NCHUNK)
        def _start_epilogue():
            pltpu.async_copy(
                lp_hbm.at[pl.ds(row0, BEAM), pl.ds(NCHUNK * CHUNK, ECHUNK)],
                buf.at[(t + 1) % 2, :, pl.ds(0, ECHUNK)],
                sems.at[(t + 1) % 2])

        @pl.when(t > 0)
        def _wait_chunk():
            pltpu.make_async_copy(
                lp_hbm.at[pl.ds(row0, BEAM), pl.ds(t * CHUNK, CHUNK)],
                buf.at[parity], sems.at[parity]).wait()

        T_val, T_idx, theta, rowmax = lax.fori_loop(
            0, BEAM,
            functools.partial(scan_row, parity=parity, col0=t * CHUNK,
                              ncols=CHUNK, sub=100, grp=10),
            (T_val, T_idx, theta, rowmax))
        return T_val, T_idx, theta, rowmax

    init = (jnp.full((16,), NEG, jnp.float32),
            jnp.zeros((16,), jnp.int32),
            jnp.float32(NEG),
            jnp.full((16,), NEG, jnp.float32))

    pltpu.make_async_copy(
        lp_hbm.at[pl.ds(row0, BEAM), pl.ds(0, CHUNK)],
        buf.at[0], sems.at[0]).wait()
    eos_vec = jnp.full((16,), NEG, jnp.float32)
    for r in range(BEAM):
        v0 = buf[0, r, pl.ds(0, 16)]
        e_r = jnp.max(jnp.where(lanes == EOS_INDEX, v0, NEG))
        eos_vec = jnp.where(lanes == r, jnp.full((16,), e_r), eos_vec)
        buf[0, r, pl.ds(0, 16)] = jnp.where(
            lanes == EOS_INDEX, jnp.full((16,), NEG, jnp.float32), v0)

    T_val, T_idx, theta, rowmax = lax.fori_loop(0, NCHUNK, step, init)

    ep = NCHUNK % 2
    pltpu.make_async_copy(
        lp_hbm.at[pl.ds(row0, BEAM), pl.ds(NCHUNK * CHUNK, ECHUNK)],
        buf.at[ep, :, pl.ds(0, ECHUNK)], sems.at[ep]).wait()
    T_val, T_idx, theta, rowmax = lax.fori_loop(
        0, BEAM,
        functools.partial(scan_row, parity=ep, col0=NCHUNK * CHUNK,
                          ncols=ECHUNK, sub=31, grp=31),
        (T_val, T_idx, theta, rowmax))

    def tail_row(r, carry):
        T_val, T_idx, theta, rowmax = carry
        s_r = s_of(r)
        v0 = tailbuf[r, pl.ds(0, 16)]
        v1 = tailbuf[r, pl.ds(16, 16)]
        m = jnp.max(jnp.maximum(v0, v1))
        rowmax = jnp.where(lanes == r, jnp.maximum(rowmax, m), rowmax)

        def do_merge(T_val, T_idx, theta):
            tv, ti = T_val, T_idx
            sb16 = jnp.full((16,), s_r)
            base = r * VOCAB + ALIGNED
            tv, ti = _merge(tv, ti, v0 + sb16, base + lanes)
            tv, ti = _merge(tv, ti, v1 + sb16, base + 16 + lanes)
            return tv, ti, _theta_of(tv)

        T_val, T_idx, theta = lax.cond(
            m + s_r > theta, do_merge,
            lambda tv, ti, th: (tv, ti, th), T_val, T_idx, theta)
        return T_val, T_idx, theta, rowmax

    T_val, T_idx, theta, rowmax = lax.fori_loop(
        0, BEAM, tail_row, (T_val, T_idx, theta, rowmax))

    true_rowmax = jnp.maximum(rowmax, eos_vec)
    keep = eos_vec > EOS_THRESHOLD * true_rowmax
    eos_vals = jnp.where(
        jnp.logical_and(lanes < BEAM, keep), eos_vec,
        jnp.where(lanes < BEAM, jnp.full((16,), jnp.float32(MINUS_INF)),
                  jnp.full((16,), NEG, jnp.float32)))
    eos_vals = jnp.where(lanes < BEAM, eos_vals + sv, eos_vals)
    eos_idx = lanes * VOCAB + EOS_INDEX
    T_val, T_idx = _merge(T_val, T_idx, eos_vals, eos_idx)

    ov[...] = T_val
    ot[...] = T_idx % VOCAB
    op[...] = T_idx // VOCAB
    pltpu.sync_copy(ov, out_s.at[pl.ds(wid * 16, 16)])
    pltpu.sync_copy(ot, out_t.at[pl.ds(wid * 16, 16)])
    pltpu.sync_copy(op, out_p.at[pl.ds(wid * 16, 16)])


@jax.jit
def kernel(log_probs, sequence_scores):
    mesh = plsc.VectorSubcoreMesh(core_axis_name="c", subcore_axis_name="s")
    f = pl.kernel(
        _body,
        mesh=mesh,
        compiler_params=pltpu.CompilerParams(needs_layout_passes=False),
        out_type=(
            jax.ShapeDtypeStruct((BATCH * 16,), jnp.float32),
            jax.ShapeDtypeStruct((BATCH * 16,), jnp.int32),
            jax.ShapeDtypeStruct((BATCH * 16,), jnp.int32),
        ),
        scratch_types=[
            pltpu.VMEM((2, BEAM, CHUNK), jnp.float32),
            pltpu.VMEM((BEAM, TAIL), jnp.float32),
            pltpu.VMEM((BATCH * BEAM + 16,), jnp.float32),
            pltpu.VMEM((16,), jnp.float32),
            pltpu.VMEM((16,), jnp.int32),
            pltpu.VMEM((16,), jnp.int32),
            pltpu.SemaphoreType.DMA((2,)),
            pltpu.SemaphoreType.DMA,
        ],
    )
    scores, toks, preds = f(log_probs, sequence_scores)
    scores = scores.reshape(BATCH, 16)[:, :BEAM]
    toks = toks.reshape(BATCH, 16)[:, :BEAM]
    preds = preds.reshape(BATCH, 16)[:, :BEAM]
    return scores, toks, preds
